# baseline (speedup 1.0000x reference)
import functools

import jax
import jax.numpy as jnp
from jax import lax
from jax.experimental import pallas as pl
from jax.experimental.pallas import tpu as pltpu
from jax.experimental.pallas import tpu_sc as plsc

NC, NS, L = 2, 16, 16
NW = NC * NS
B, T, D = 16384, 50, 64
ROW_W = T * D
ROWS_W = B // NW
R_BLK = 8
NBLK = ROWS_W // R_BLK
IDX_CH = 128
N_CH = ROWS_W // IDX_CH


def _sc_call(x2, sid3, comb_t):
    mesh = plsc.VectorSubcoreMesh(core_axis_name="c", subcore_axis_name="s",
                                  num_cores=NC)

    @functools.partial(
        pl.kernel,
        out_type=jax.ShapeDtypeStruct((B, ROW_W), jnp.float32),
        mesh=mesh,
        scratch_types=[
            pltpu.VMEM((N_CH, IDX_CH), jnp.int32),
            pltpu.VMEM((ROWS_W, 2 * D), jnp.float32),
            pltpu.VMEM((R_BLK, ROW_W), jnp.float32),
            pltpu.VMEM((R_BLK, ROW_W), jnp.float32),
            pltpu.SemaphoreType.DMA,
        ],
    )
    def k(x_hbm, sid_hbm, comb_hbm, out_hbm,
          idx_v, coef_v, in_v, out_v, gsem):
        wid = lax.axis_index("s") * NC + lax.axis_index("c")
        base_row = wid * ROWS_W

        pltpu.sync_copy(sid_hbm.at[wid], idx_v)
        copies = []
        for j in range(N_CH):
            copies.append(pltpu.async_copy(
                comb_hbm.at[idx_v.at[j]],
                coef_v.at[pl.ds(j * IDX_CH, IDX_CH)], gsem))
        for c in copies:
            c.wait()

        def block_body(g, carry):
            r0 = base_row + g * R_BLK
            pltpu.sync_copy(x_hbm.at[pl.ds(r0, R_BLK)], in_v)
            for r in range(R_BLK):
                row = g * R_BLK + r
                sc = [coef_v[row, pl.ds(L * j, L)] for j in range(D // L)]
                sh = [coef_v[row, pl.ds(D + L * j, L)] for j in range(D // L)]

                def t_body(t, c2):
                    col = t * D
                    for j in range(D // L):
                        cj = col + L * j
                        out_v[r, pl.ds(cj, L)] = (
                            in_v[r, pl.ds(cj, L)] * sc[j] + sh[j])
                    return c2

                lax.fori_loop(0, T, t_body, 0)
            pltpu.sync_copy(out_v, out_hbm.at[pl.ds(r0, R_BLK)])
            return carry

        lax.fori_loop(0, NBLK, block_body, 0)

    return k(x2, sid3, comb_t)


def kernel(x, session_id, session_shift, session_scale):
    x2 = x.reshape(B, ROW_W)
    sid3 = session_id.astype(jnp.int32).reshape(NW, N_CH, IDX_CH)
    comb = jnp.concatenate([session_scale, session_shift], axis=1)
    out = _sc_call(x2, sid3, comb)
    return out.reshape(B, T, D)


# speedup vs baseline: 1.7109x; 1.7109x over previous
import functools

import jax
import jax.numpy as jnp
from jax import lax
from jax.experimental import pallas as pl
from jax.experimental.pallas import tpu as pltpu
from jax.experimental.pallas import tpu_sc as plsc

NC, NS, L = 2, 16, 16
NW = NC * NS
B, T, D = 16384, 50, 64
ROW_W = T * D
ROWS_W = B // NW
R_BLK = 8
NBLK = ROWS_W // R_BLK
NPAD = 128


def _sc_call(x2, sid2, comb_t):
    mesh = plsc.VectorSubcoreMesh(core_axis_name="c", subcore_axis_name="s",
                                  num_cores=NC)

    @functools.partial(
        pl.kernel,
        out_type=jax.ShapeDtypeStruct((B, ROW_W), jnp.float32),
        mesh=mesh,
        scratch_types=[
            pltpu.VMEM((ROWS_W,), jnp.int32),
            pltpu.VMEM((NPAD, 2 * D), jnp.float32),
            pltpu.VMEM((R_BLK, ROW_W), jnp.float32),
            pltpu.VMEM((R_BLK, ROW_W), jnp.float32),
            pltpu.VMEM((R_BLK, ROW_W), jnp.float32),
            pltpu.VMEM((R_BLK, ROW_W), jnp.float32),
            pltpu.SemaphoreType.DMA,
            pltpu.SemaphoreType.DMA,
            pltpu.SemaphoreType.DMA,
            pltpu.SemaphoreType.DMA,
        ],
    )
    def k(x_hbm, sid_hbm, comb_hbm, out_hbm,
          idx_v, tab_v, in0, in1, ou0, ou1, l0, l1, s0, s1):
        wid = lax.axis_index("s") * NC + lax.axis_index("c")
        base_row = wid * ROWS_W
        ins, ous = (in0, in1), (ou0, ou1)
        lsem, ssem = (l0, l1), (s0, s1)

        pltpu.sync_copy(sid_hbm.at[wid], idx_v)
        pltpu.sync_copy(comb_hbm, tab_v)
        pltpu.async_copy(x_hbm.at[pl.ds(base_row, R_BLK)], in0, l0)
        pltpu.async_copy(x_hbm.at[pl.ds(base_row + R_BLK, R_BLK)], in1, l1)

        lane = lax.iota(jnp.int32, L)

        def step(h, carry):
            for kk in range(2):
                g = h * 2 + kk
                inb, oub = ins[kk], ous[kk]

                pltpu.make_async_copy(
                    x_hbm.at[pl.ds(base_row, R_BLK)], inb, lsem[kk]).wait()

                @pl.when(g >= 2)
                def _wait_store():
                    pltpu.make_async_copy(
                        x_hbm.at[pl.ds(base_row, R_BLK)], oub,
                        ssem[kk]).wait()

                for r in range(R_BLK):
                    row = g * R_BLK + r
                    sid = plsc.load_gather(
                        idx_v, [jnp.full((L,), row, jnp.int32)])
                    sc = [plsc.load_gather(tab_v, [sid, lane + L * j])
                          for j in range(D // L)]
                    sh = [plsc.load_gather(tab_v, [sid, lane + D + L * j])
                          for j in range(D // L)]

                    @functools.partial(plsc.parallel_loop, 0, T, unroll=2)
                    def _t_body(t):
                        col = t * D
                        for j in range(D // L):
                            cj = col + L * j
                            oub[r, pl.ds(cj, L)] = (
                                inb[r, pl.ds(cj, L)] * sc[j] + sh[j])

                r0 = base_row + g * R_BLK
                pltpu.async_copy(oub, out_hbm.at[pl.ds(r0, R_BLK)], ssem[kk])

                @pl.when(g + 2 < NBLK)
                def _next_load():
                    rn = base_row + (g + 2) * R_BLK
                    pltpu.async_copy(
                        x_hbm.at[pl.ds(rn, R_BLK)], inb, lsem[kk])
            return carry

        lax.fori_loop(0, NBLK // 2, step, 0)

        pltpu.make_async_copy(x_hbm.at[pl.ds(base_row, R_BLK)], ou0, s0).wait()
        pltpu.make_async_copy(x_hbm.at[pl.ds(base_row, R_BLK)], ou1, s1).wait()

    return k(x2, sid2, comb_t)


def kernel(x, session_id, session_shift, session_scale):
    x2 = x.reshape(B, ROW_W)
    sid2 = session_id.astype(jnp.int32).reshape(NW, ROWS_W)
    comb = jnp.concatenate([session_scale, session_shift], axis=1)
    n = comb.shape[0]
    comb = jnp.pad(comb, ((0, NPAD - n), (0, 0)))
    out = _sc_call(x2, sid2, comb)
    return out.reshape(B, T, D)


# speedup vs baseline: 1.7143x; 1.0020x over previous
import functools

import jax
import jax.numpy as jnp
from jax import lax
from jax.experimental import pallas as pl
from jax.experimental.pallas import tpu as pltpu
from jax.experimental.pallas import tpu_sc as plsc

NC, NS, L = 2, 16, 16
NW = NC * NS
B, T, D = 16384, 50, 64
ROW_W = T * D
ROWS_W = B // NW
R_BLK = 8
NBLK = ROWS_W // R_BLK
NPAD = 128


def _sc_call(x2, sid2, comb_t):
    mesh = plsc.VectorSubcoreMesh(core_axis_name="c", subcore_axis_name="s",
                                  num_cores=NC)

    @functools.partial(
        pl.kernel,
        out_type=jax.ShapeDtypeStruct((B, ROW_W), jnp.float32),
        mesh=mesh,
        scratch_types=[
            pltpu.VMEM((ROWS_W,), jnp.int32),
            pltpu.VMEM((NPAD, 2 * D), jnp.float32),
            pltpu.VMEM((R_BLK, ROW_W), jnp.float32),
            pltpu.VMEM((R_BLK, ROW_W), jnp.float32),
            pltpu.VMEM((R_BLK, ROW_W), jnp.float32),
            pltpu.VMEM((R_BLK, ROW_W), jnp.float32),
            pltpu.SemaphoreType.DMA,
            pltpu.SemaphoreType.DMA,
            pltpu.SemaphoreType.DMA,
            pltpu.SemaphoreType.DMA,
        ],
    )
    def k(x_hbm, sid_hbm, comb_hbm, out_hbm,
          idx_v, tab_v, in0, in1, ou0, ou1, l0, l1, s0, s1):
        wid = lax.axis_index("s") * NC + lax.axis_index("c")
        base_row = wid * ROWS_W
        ins, ous = (in0, in1), (ou0, ou1)
        lsem, ssem = (l0, l1), (s0, s1)

        pltpu.sync_copy(sid_hbm.at[wid], idx_v)
        pltpu.sync_copy(comb_hbm, tab_v)
        pltpu.async_copy(x_hbm.at[pl.ds(base_row, R_BLK)], in0, l0)
        pltpu.async_copy(x_hbm.at[pl.ds(base_row + R_BLK, R_BLK)], in1, l1)

        lane = lax.iota(jnp.int32, L)

        def step(h, carry):
            for kk in range(2):
                g = h * 2 + kk
                inb, oub = ins[kk], ous[kk]

                pltpu.make_async_copy(
                    x_hbm.at[pl.ds(base_row, R_BLK)], inb, lsem[kk]).wait()

                @pl.when(g >= 2)
                def _wait_store():
                    pltpu.make_async_copy(
                        x_hbm.at[pl.ds(base_row, R_BLK)], oub,
                        ssem[kk]).wait()

                for r in range(R_BLK):
                    row = g * R_BLK + r
                    sid = plsc.load_gather(
                        idx_v, [jnp.full((L,), row, jnp.int32)])
                    sc = [plsc.load_gather(tab_v, [sid, lane + L * j])
                          for j in range(D // L)]
                    sh = [plsc.load_gather(tab_v, [sid, lane + D + L * j])
                          for j in range(D // L)]

                    @functools.partial(plsc.parallel_loop, 0, T, unroll=5)
                    def _t_body(t):
                        col = t * D
                        for j in range(D // L):
                            cj = col + L * j
                            oub[r, pl.ds(cj, L)] = (
                                inb[r, pl.ds(cj, L)] * sc[j] + sh[j])

                r0 = base_row + g * R_BLK
                pltpu.async_copy(oub, out_hbm.at[pl.ds(r0, R_BLK)], ssem[kk])

                @pl.when(g + 2 < NBLK)
                def _next_load():
                    rn = base_row + (g + 2) * R_BLK
                    pltpu.async_copy(
                        x_hbm.at[pl.ds(rn, R_BLK)], inb, lsem[kk])
            return carry

        lax.fori_loop(0, NBLK // 2, step, 0)

        pltpu.make_async_copy(x_hbm.at[pl.ds(base_row, R_BLK)], ou0, s0).wait()
        pltpu.make_async_copy(x_hbm.at[pl.ds(base_row, R_BLK)], ou1, s1).wait()

    return k(x2, sid2, comb_t)


def kernel(x, session_id, session_shift, session_scale):
    x2 = x.reshape(B, ROW_W)
    sid2 = session_id.astype(jnp.int32).reshape(NW, ROWS_W)
    comb = jnp.concatenate([session_scale, session_shift], axis=1)
    n = comb.shape[0]
    comb = jnp.pad(comb, ((0, NPAD - n), (0, 0)))
    out = _sc_call(x2, sid2, comb)
    return out.reshape(B, T, D)
